# Initial kernel scaffold; baseline (speedup 1.0000x reference)
import functools

import jax
import jax.numpy as jnp
from jax import lax
from jax.experimental import pallas as pl
from jax.experimental.pallas import tpu as pltpu
from jax.experimental.pallas import tpu_sc as plsc

_NC = 2
_NS = 16
_NW = _NC * _NS

_GATHER = 128
_GPG = 5
_GROUP = _GATHER * _GPG


def _table_body(tbl_ref, w_ref, b_ref, o_ref):
    tbl = tbl_ref[...]
    rid = lax.broadcasted_iota(jnp.int32, tbl.shape, 0)
    tbl = jnp.where(rid == 0, jnp.float32(0.0), tbl)
    y = lax.dot_general(tbl, w_ref[...], (((1,), (1,)), ((), ())),
                        preferred_element_type=jnp.float32)
    o_ref[...] = jnp.tanh(y + b_ref[...])


def _transform_table(emb_table, W, b):
    n, d = emb_table.shape
    return pl.pallas_call(
        _table_body,
        out_shape=jax.ShapeDtypeStruct((n, d), jnp.float32),
    )(emb_table, W, b.reshape(1, d))


def _gather_body(n_super, d, table_hbm, idx_hbm, out_hbm,
                 idx_v, rows_v, gsem, ssem0, ssem1):
    wid = lax.axis_index("s") * _NC + lax.axis_index("c")
    rows_per_w = n_super * 2 * _GPG
    base_row = wid * rows_per_w
    base_out = base_row * _GATHER
    ssems = (ssem0, ssem1)

    def super_body(t, carry):
        for s in range(2):
            g = 2 * t + s

            @pl.when(t > 0)
            def _wait_store(s=s):
                pltpu.make_async_copy(
                    rows_v.at[s], out_hbm.at[pl.ds(0, _GROUP)], ssems[s]
                ).wait()

            pltpu.sync_copy(
                idx_hbm.at[pl.ds(base_row + g * _GPG, _GPG)], idx_v.at[s])
            cps = [
                pltpu.async_copy(
                    table_hbm.at[idx_v.at[s, j]],
                    rows_v.at[s, pl.ds(j * _GATHER, _GATHER)],
                    gsem,
                )
                for j in range(_GPG)
            ]
            for cp in cps:
                cp.wait()
            pltpu.async_copy(
                rows_v.at[s],
                out_hbm.at[pl.ds(base_out + g * _GROUP, _GROUP)],
                ssems[s],
            )
        return carry

    lax.fori_loop(0, n_super, super_body, 0)
    for s in range(2):
        pltpu.make_async_copy(
            rows_v.at[s], out_hbm.at[pl.ds(0, _GROUP)], ssems[s]).wait()


def _gather_rows(new_table, idx2d):
    n_rows = idx2d.shape[0]
    d = new_table.shape[1]
    total = n_rows * _GATHER
    assert n_rows % (_NW * 2 * _GPG) == 0
    n_super = n_rows // (_NW * 2 * _GPG)

    mesh = plsc.VectorSubcoreMesh(core_axis_name="c", subcore_axis_name="s")
    kern = pl.kernel(
        functools.partial(_gather_body, n_super, d),
        mesh=mesh,
        out_type=jax.ShapeDtypeStruct((total, d), jnp.float32),
        scratch_types=[
            pltpu.VMEM((2, _GPG, _GATHER), jnp.int32),
            pltpu.VMEM((2, _GROUP, d), jnp.float32),
            pltpu.SemaphoreType.DMA,
            pltpu.SemaphoreType.DMA,
            pltpu.SemaphoreType.DMA,
        ],
    )
    return kern(new_table, idx2d)


def kernel(sentiment, emb_table, W, b):
    batch, hist = sentiment.shape
    d = emb_table.shape[1]
    new_table = _transform_table(emb_table, W, b)
    idx2d = sentiment.reshape(batch * hist // _GATHER, _GATHER)
    out = _gather_rows(new_table, idx2d)
    return out.reshape(batch, hist, d)


# speedup vs baseline: 4.0523x; 4.0523x over previous
import functools

import jax
import jax.numpy as jnp
from jax import lax
from jax.experimental import pallas as pl
from jax.experimental.pallas import tpu as pltpu
from jax.experimental.pallas import tpu_sc as plsc

_NC = 2
_NS = 16
_NW = _NC * _NS

_GATHER = 128
_GPG = 5
_GROUP = _GATHER * _GPG


def _table_body(tbl_ref, w_ref, b_ref, o_ref):
    tbl = tbl_ref[...]
    rid = lax.broadcasted_iota(jnp.int32, tbl.shape, 0)
    tbl = jnp.where(rid == 0, jnp.float32(0.0), tbl)
    y = lax.dot_general(tbl, w_ref[...], (((1,), (1,)), ((), ())),
                        preferred_element_type=jnp.float32)
    o_ref[...] = jnp.tanh(y + b_ref[...])


def _transform_table(emb_table, W, b):
    n, d = emb_table.shape
    return pl.pallas_call(
        _table_body,
        out_shape=jax.ShapeDtypeStruct((n, d), jnp.float32),
    )(emb_table, W, b.reshape(1, d))


def _gather_body(n_super, d, table_hbm, idx_hbm, out_hbm,
                 idx_v, rows_v, gsem, ssem0, ssem1):
    wid = lax.axis_index("s") * _NC + lax.axis_index("c")
    per_w = n_super * 2 * _GROUP
    base_out = wid * per_w
    ssems = (ssem0, ssem1)

    def super_body(t, carry):
        for s in range(2):
            g = 2 * t + s

            @pl.when(t > 0)
            def _wait_store(s=s):
                pltpu.make_async_copy(
                    rows_v.at[s], out_hbm.at[pl.ds(0, _GROUP)], ssems[s]
                ).wait()

            pltpu.sync_copy(
                idx_hbm.at[pl.ds(base_out + g * _GROUP, _GROUP)], idx_v.at[s])
            cps = [
                pltpu.async_copy(
                    table_hbm.at[idx_v.at[s, pl.ds(j * _GATHER, _GATHER)]],
                    rows_v.at[s, pl.ds(j * _GATHER, _GATHER)],
                    gsem,
                )
                for j in range(_GPG)
            ]
            for cp in cps:
                cp.wait()
            pltpu.async_copy(
                rows_v.at[s],
                out_hbm.at[pl.ds(base_out + g * _GROUP, _GROUP)],
                ssems[s],
            )
        return carry

    lax.fori_loop(0, n_super, super_body, 0)
    for s in range(2):
        pltpu.make_async_copy(
            rows_v.at[s], out_hbm.at[pl.ds(0, _GROUP)], ssems[s]).wait()


def _gather_rows(new_table, idx):
    total = idx.shape[0]
    d = new_table.shape[1]
    assert total % (_NW * 2 * _GROUP) == 0
    n_super = total // (_NW * 2 * _GROUP)

    mesh = plsc.VectorSubcoreMesh(core_axis_name="c", subcore_axis_name="s")
    kern = pl.kernel(
        functools.partial(_gather_body, n_super, d),
        mesh=mesh,
        compiler_params=pltpu.CompilerParams(use_tc_tiling_on_sc=False),
        out_type=jax.ShapeDtypeStruct((total, d), jnp.float32),
        scratch_types=[
            pltpu.VMEM((2, _GROUP), jnp.int32),
            pltpu.VMEM((2, _GROUP, d), jnp.float32),
            pltpu.SemaphoreType.DMA,
            pltpu.SemaphoreType.DMA,
            pltpu.SemaphoreType.DMA,
        ],
    )
    return kern(new_table, idx)


def kernel(sentiment, emb_table, W, b):
    batch, hist = sentiment.shape
    d = emb_table.shape[1]
    new_table = _transform_table(emb_table, W, b)
    out = _gather_rows(new_table, sentiment.reshape(-1))
    return out.reshape(batch, hist, d)
